# baseline (speedup 1.0000x reference)
import functools

import jax
import jax.numpy as jnp
from jax import lax
from jax.experimental import pallas as pl
from jax.experimental.pallas import tpu as pltpu
from jax.experimental.pallas import tpu_sc as plsc

_B = 4
_S = 8192
_D = 1024
_N = _S * _D
_NC, _NS, _L = 2, 16, 16
_NW = _NC * _NS
_CHUNK = _N // _NW
_C = 16384
_UNROLL = 8


def _sc_body(x_hbm, t_hbm, out_hbm, t_v, x_v):
    wid = lax.axis_index("s") * _NC + lax.axis_index("c")
    base = wid * _CHUNK

    def outer(j, carry):
        off = base + j * _C
        pltpu.sync_copy(t_hbm.at[pl.ds(off, _C)], t_v)

        def per_batch(b, carry2):
            pltpu.sync_copy(x_hbm.at[b, pl.ds(off, _C)], x_v)

            def inner(i, carry3):
                for u in range(_UNROLL):
                    s = pl.ds((i * _UNROLL + u) * _L, _L)
                    x_v[s] = x_v[s] + t_v[s]
                return carry3

            lax.fori_loop(0, _C // (_L * _UNROLL), inner, 0)
            pltpu.sync_copy(x_v, out_hbm.at[b, pl.ds(off, _C)])
            return carry2

        lax.fori_loop(0, _B, per_batch, 0)
        return carry

    lax.fori_loop(0, _CHUNK // _C, outer, 0)


@jax.jit
def _sc_add(x2, t1):
    mesh = plsc.VectorSubcoreMesh(core_axis_name="c", subcore_axis_name="s")
    f = functools.partial(
        pl.kernel,
        mesh=mesh,
        out_type=jax.ShapeDtypeStruct((_B, _N), jnp.float32),
        scratch_types=[
            pltpu.VMEM((_C,), jnp.float32),
            pltpu.VMEM((_C,), jnp.float32),
        ],
    )(_sc_body)
    return f(x2, t1)


def kernel(x, table):
    B, S, D = x.shape
    out2 = _sc_add(x.reshape(B, S * D), table.reshape(S * D))
    return out2.reshape(B, S, D)


# speedup vs baseline: 1.3470x; 1.3470x over previous
import functools

import jax
import jax.numpy as jnp
from jax import lax
from jax.experimental import pallas as pl
from jax.experimental.pallas import tpu as pltpu
from jax.experimental.pallas import tpu_sc as plsc

_B = 4
_S = 8192
_D = 1024
_N = _S * _D
_NC, _NS, _L = 2, 16, 16
_NW = _NC * _NS
_CHUNK = _N // _NW
_C = 16384
_NJ = _CHUNK // _C
_NT = _B * _NJ
_UNROLL = 8


def _sc_body(x_hbm, t_hbm, out_hbm,
             xb0, xb1, xb2, xb3, tb0, tb1,
             si0, si1, si2, si3, so0, so1, so2, so3, st0, st1):
    xbs = (xb0, xb1, xb2, xb3)
    sis = (si0, si1, si2, si3)
    sos = (so0, so1, so2, so3)
    tbs = (tb0, tb1)
    sts = (st0, st1)

    wid = lax.axis_index("s") * _NC + lax.axis_index("c")
    base = wid * _CHUNK

    def start_in(j, batch, bufi):
        pltpu.async_copy(
            x_hbm.at[batch, pl.ds(base + j * _C, _C)], xbs[bufi], sis[bufi])

    def wait_in(bufi):
        pltpu.make_async_copy(
            x_hbm.at[0, pl.ds(base, _C)], xbs[bufi], sis[bufi]).wait()

    def start_t(j, ti):
        pltpu.async_copy(t_hbm.at[pl.ds(base + j * _C, _C)], tbs[ti], sts[ti])

    def wait_t(ti):
        pltpu.make_async_copy(
            t_hbm.at[pl.ds(base, _C)], tbs[ti], sts[ti]).wait()

    def start_out(j, batch, bufi):
        pltpu.async_copy(
            xbs[bufi], out_hbm.at[batch, pl.ds(base + j * _C, _C)], sos[bufi])

    def wait_out(bufi):
        pltpu.make_async_copy(
            xbs[bufi], out_hbm.at[0, pl.ds(base, _C)], sos[bufi]).wait()

    start_t(0, 0)
    start_in(0, 0, 0)
    start_in(0, 1, 1)

    def outer(g, carry):
        for m in range(8):
            jj, b = divmod(m, 4)
            j = 2 * g + jj
            wait_in(b)
            if b == 0:
                wait_t(jj)

            def _add(i, c3):
                for u in range(_UNROLL):
                    s = pl.ds((i * _UNROLL + u) * _L, _L)
                    xbs[b][s] = xbs[b][s] + tbs[jj][s]
                return c3

            lax.fori_loop(0, _C // (_L * _UNROLL), _add, 0)

            start_out(j, b, b)
            if b == 0:
                if m == 0:
                    start_t(j + 1, 1)
                else:
                    @pl.when(g < (_NJ // 2) - 1)
                    def _():
                        start_t(j + 1, 0)
            b2 = (m + 2) % 4
            if m < 2:
                @pl.when(g > 0)
                def _():
                    wait_out(b2)
            else:
                wait_out(b2)
            if m < 6:
                start_in(2 * g + (m + 2) // 4, b2, b2)
            else:
                @pl.when(g < (_NJ // 2) - 1)
                def _():
                    start_in(2 * (g + 1), b2, b2)
        return carry

    lax.fori_loop(0, _NJ // 2, outer, 0)
    wait_out(2)
    wait_out(3)


@jax.jit
def _sc_add(x2, t1):
    mesh = plsc.VectorSubcoreMesh(core_axis_name="c", subcore_axis_name="s")
    f = functools.partial(
        pl.kernel,
        mesh=mesh,
        out_type=jax.ShapeDtypeStruct((_B, _N), jnp.float32),
        scratch_types=(
            [pltpu.VMEM((_C,), jnp.float32)] * 4
            + [pltpu.VMEM((_C,), jnp.float32)] * 2
            + [pltpu.SemaphoreType.DMA] * 10
        ),
    )(_sc_body)
    return f(x2, t1)


def kernel(x, table):
    B, S, D = x.shape
    out2 = _sc_add(x.reshape(B, S * D), table.reshape(S * D))
    return out2.reshape(B, S, D)


# speedup vs baseline: 1.3961x; 1.0365x over previous
import functools

import jax
import jax.numpy as jnp
from jax import lax
from jax.experimental import pallas as pl
from jax.experimental.pallas import tpu as pltpu
from jax.experimental.pallas import tpu_sc as plsc

_B = 4
_S = 8192
_D = 1024
_N = _S * _D
_NC, _NS, _L = 2, 16, 16
_NW = _NC * _NS
_K = 1024
_KN = _K * _D
_CHUNK = _KN // _NW
_C = 16384
_NJ = _CHUNK // _C
_UNROLL = 8
_BS = 512


def _sc_body(x_hbm, t_hbm, out_hbm,
             xb0, xb1, xb2, xb3, tb0, tb1,
             si0, si1, si2, si3, so0, so1, so2, so3, st0, st1):
    xbs = (xb0, xb1, xb2, xb3)
    sis = (si0, si1, si2, si3)
    sos = (so0, so1, so2, so3)
    tbs = (tb0, tb1)
    sts = (st0, st1)

    wid = lax.axis_index("s") * _NC + lax.axis_index("c")
    base = wid * _CHUNK

    def start_in(j, batch, bufi):
        pltpu.async_copy(
            x_hbm.at[batch, pl.ds(base + j * _C, _C)], xbs[bufi], sis[bufi])

    def wait_in(bufi):
        pltpu.make_async_copy(
            x_hbm.at[0, pl.ds(base, _C)], xbs[bufi], sis[bufi]).wait()

    def start_t(j, ti):
        pltpu.async_copy(t_hbm.at[pl.ds(base + j * _C, _C)], tbs[ti], sts[ti])

    def wait_t(ti):
        pltpu.make_async_copy(
            t_hbm.at[pl.ds(base, _C)], tbs[ti], sts[ti]).wait()

    def start_out(j, batch, bufi):
        pltpu.async_copy(
            xbs[bufi], out_hbm.at[batch, pl.ds(base + j * _C, _C)], sos[bufi])

    def wait_out(bufi):
        pltpu.make_async_copy(
            xbs[bufi], out_hbm.at[0, pl.ds(base, _C)], sos[bufi]).wait()

    start_t(0, 0)
    start_in(0, 0, 0)
    start_in(0, 1, 1)

    def outer(g, carry):
        for m in range(8):
            jj, b = divmod(m, 4)
            j = 2 * g + jj
            wait_in(b)
            if b == 0:
                wait_t(jj)

            def _add(i, c3):
                for u in range(_UNROLL):
                    s = pl.ds((i * _UNROLL + u) * _L, _L)
                    xbs[b][s] = xbs[b][s] + tbs[jj][s]
                return c3

            lax.fori_loop(0, _C // (_L * _UNROLL), _add, 0)
            start_out(j, b, b)
            if b == 0:
                if m == 0:
                    @pl.when(2 * g + 1 < _NJ)
                    def _():
                        start_t(j + 1, 1)
                else:
                    @pl.when(g < (_NJ // 2) - 1)
                    def _():
                        start_t(j + 1, 0)
            b2 = (m + 2) % 4
            if m < 2:
                @pl.when(g > 0)
                def _():
                    wait_out(b2)
            else:
                wait_out(b2)
            if m < 6:
                start_in(2 * g + (m + 2) // 4, b2, b2)
            else:
                @pl.when(g < (_NJ // 2) - 1)
                def _():
                    start_in(2 * (g + 1), b2, b2)
        return carry

    lax.fori_loop(0, _NJ // 2, outer, 0)
    wait_out(2)
    wait_out(3)


def _sc_add(x2, t1):
    mesh = plsc.VectorSubcoreMesh(core_axis_name="c", subcore_axis_name="s")
    f = functools.partial(
        pl.kernel,
        mesh=mesh,
        out_type=jax.ShapeDtypeStruct((_B, _KN), jnp.float32),
        scratch_types=(
            [pltpu.VMEM((_C,), jnp.float32)] * 4
            + [pltpu.VMEM((_C,), jnp.float32)] * 2
            + [pltpu.SemaphoreType.DMA] * 10
        ),
    )(_sc_body)
    return f(x2, t1)


def _tc_body(x_ref, t_ref, o_ref):
    o_ref[...] = x_ref[...] + t_ref[...][None, :, :]


def _tc_add(x, table):
    koff = _K // _BS
    return pl.pallas_call(
        _tc_body,
        grid=((_S - _K) // _BS,),
        in_specs=[
            pl.BlockSpec((_B, _BS, _D), lambda i: (0, i + koff, 0)),
            pl.BlockSpec((_BS, _D), lambda i: (i + koff, 0)),
        ],
        out_specs=pl.BlockSpec((_B, _BS, _D), lambda i: (0, i, 0)),
        out_shape=jax.ShapeDtypeStruct((_B, _S - _K, _D), x.dtype),
    )(x, table)


@jax.jit
def _hybrid(x, table):
    sc_out = _sc_add(x.reshape(_B, _N), table.reshape(_N))
    tc_out = _tc_add(x, table)
    return jnp.concatenate([sc_out.reshape(_B, _K, _D), tc_out], axis=1)


def kernel(x, table):
    return _hybrid(x, table)


# speedup vs baseline: 3.2304x; 2.3138x over previous
import functools

import jax
import jax.numpy as jnp
from jax import lax
from jax.experimental import pallas as pl
from jax.experimental.pallas import tpu as pltpu
from jax.experimental.pallas import tpu_sc as plsc

_B = 4
_S = 8192
_D = 1024
_NC, _NS, _L = 2, 16, 16
_NW = _NC * _NS
_ROWS = _S // _NW
_R = 16
_NJ = _ROWS // _R


def _sc_body(x_hbm, t_hbm, out_hbm,
             xb0, xb1, xb2, xb3, tb0, tb1,
             si0, si1, si2, si3, so0, so1, so2, so3, st0, st1):
    xbs = (xb0, xb1, xb2, xb3)
    sis = (si0, si1, si2, si3)
    sos = (so0, so1, so2, so3)
    tbs = (tb0, tb1)
    sts = (st0, st1)

    wid = lax.axis_index("s") * _NC + lax.axis_index("c")
    base = wid * _ROWS

    def start_in(j, batch, bufi):
        pltpu.async_copy(
            x_hbm.at[batch, pl.ds(base + j * _R, _R), :], xbs[bufi], sis[bufi])

    def wait_in(bufi):
        pltpu.make_async_copy(
            x_hbm.at[0, pl.ds(base, _R), :], xbs[bufi], sis[bufi]).wait()

    def start_t(j, ti):
        pltpu.async_copy(
            t_hbm.at[pl.ds(base + j * _R, _R), :], tbs[ti], sts[ti])

    def wait_t(ti):
        pltpu.make_async_copy(
            t_hbm.at[pl.ds(base, _R), :], tbs[ti], sts[ti]).wait()

    def start_out(j, batch, bufi):
        pltpu.async_copy(
            xbs[bufi], out_hbm.at[batch, pl.ds(base + j * _R, _R), :],
            sos[bufi])

    def wait_out(bufi):
        pltpu.make_async_copy(
            xbs[bufi], out_hbm.at[0, pl.ds(base, _R), :], sos[bufi]).wait()

    start_t(0, 0)
    start_in(0, 0, 0)
    start_in(0, 1, 1)

    def outer(g, carry):
        for m in range(8):
            jj, b = divmod(m, 4)
            j = 2 * g + jj
            wait_in(b)
            if b == 0:
                wait_t(jj)

            def _add(r, c3):
                for u in range(_D // _L):
                    s = pl.ds(u * _L, _L)
                    xbs[b][r, s] = xbs[b][r, s] + tbs[jj][r, s]
                return c3

            lax.fori_loop(0, _R, _add, 0)
            start_out(j, b, b)
            if b == 0:
                if m == 0:
                    @pl.when(2 * g + 1 < _NJ)
                    def _():
                        start_t(j + 1, 1)
                else:
                    @pl.when(g < (_NJ // 2) - 1)
                    def _():
                        start_t(j + 1, 0)
            b2 = (m + 2) % 4
            if m < 2:
                @pl.when(g > 0)
                def _():
                    wait_out(b2)
            else:
                wait_out(b2)
            if m < 6:
                start_in(2 * g + (m + 2) // 4, b2, b2)
            else:
                @pl.when(g < (_NJ // 2) - 1)
                def _():
                    start_in(2 * (g + 1), b2, b2)
        return carry

    lax.fori_loop(0, _NJ // 2, outer, 0)
    wait_out(2)
    wait_out(3)


@jax.jit
def _sc_add(x, table):
    mesh = plsc.VectorSubcoreMesh(core_axis_name="c", subcore_axis_name="s")
    f = functools.partial(
        pl.kernel,
        mesh=mesh,
        out_type=jax.ShapeDtypeStruct((_B, _S, _D), jnp.float32),
        scratch_types=(
            [pltpu.VMEM((_R, _D), jnp.float32)] * 4
            + [pltpu.VMEM((_R, _D), jnp.float32)] * 2
            + [pltpu.SemaphoreType.DMA] * 10
        ),
    )(_sc_body)
    return f(x, table)


def kernel(x, table):
    return _sc_add(x, table)
